# baseline (speedup 1.0000x reference)
import functools

import jax
import jax.numpy as jnp
from jax import lax
from jax.experimental import pallas as pl
from jax.experimental.pallas import tpu as pltpu
from jax.experimental.pallas import tpu_sc as plsc


def _argmax_reduce(s, io_ref, K):
    m = jnp.max(s, axis=1, keepdims=True)
    idxf = jnp.min(jnp.where(s == m, io_ref[...], float(K)), axis=1)
    return jnp.minimum(idxf, K - 1).astype(jnp.int32)


def _argmax_body(x_ref, cbT_ref, xn_ref, ide_ref, ido_ref,
                 bs_ref, io_ref, sb0_ref, sb1_ref):
    K = cbT_ref.shape[1]
    h = sb0_ref.shape[0]

    @pl.when(pl.program_id(0) == 0)
    def _prep():
        c = cbT_ref[...]
        cn = c / (jnp.sqrt(jnp.sum(c * c, axis=0, keepdims=True)) + 1e-8)
        bs_ref[...] = cn.astype(jnp.bfloat16)
        io_ref[...] = lax.broadcasted_iota(jnp.int32, (1, K),
                                           1).astype(jnp.float32)

    x = x_ref[...]
    xn = x / (jnp.sqrt(jnp.sum(x * x, axis=1, keepdims=True)) + 1e-8)
    xn_ref[...] = xn
    xb = xn.astype(jnp.bfloat16)

    sb0_ref[...] = lax.dot_general(xb[:h], bs_ref[...],
                                   (((1,), (0,)), ((), ())),
                                   preferred_element_type=jnp.float32)
    ido_ref[...] = _argmax_reduce(sb1_ref[...], io_ref, K)
    sb1_ref[...] = lax.dot_general(xb[h:], bs_ref[...],
                                   (((1,), (0,)), ((), ())),
                                   preferred_element_type=jnp.float32)
    ide_ref[...] = _argmax_reduce(sb0_ref[...], io_ref, K)


def _tc_argmax(x_DL, codebook_KL, block_d):
    D, L = x_DL.shape
    K = codebook_KL.shape[0]
    h = block_d // 2
    nj = D // block_d
    xn, ide, ido = pl.pallas_call(
        _argmax_body,
        grid=(nj + 1,),
        in_specs=[
            pl.BlockSpec((block_d, L), lambda j: (jnp.minimum(j, nj - 1), 0)),
            pl.BlockSpec((L, K), lambda j: (0, 0)),
        ],
        out_specs=[
            pl.BlockSpec((block_d, L), lambda j: (jnp.minimum(j, nj - 1), 0)),
            pl.BlockSpec((h,), lambda j: (jnp.minimum(j, nj - 1),)),
            pl.BlockSpec((h,), lambda j: (jnp.maximum(j - 1, 0),)),
        ],
        out_shape=[
            jax.ShapeDtypeStruct((D, L), jnp.float32),
            jax.ShapeDtypeStruct((D // 2,), jnp.int32),
            jax.ShapeDtypeStruct((D // 2,), jnp.int32),
        ],
        scratch_shapes=[pltpu.VMEM((L, K), jnp.bfloat16),
                        pltpu.VMEM((1, K), jnp.float32),
                        pltpu.VMEM((h, K), jnp.float32),
                        pltpu.VMEM((h, K), jnp.float32)],
    )(x_DL, codebook_KL.T)
    return xn, ide, ido, h


_SC_CHUNK = 128


def _sc_gather(codebook_KL, ide, ido, half):
    D = ide.shape[0] + ido.shape[0]
    L = codebook_KL.shape[1]
    info = plsc.get_sparse_core_info()
    nw = info.num_cores * info.num_subcores
    b_per_w = D // nw
    n_chunks = b_per_w // _SC_CHUNK
    per_half = half // _SC_CHUNK
    mesh = plsc.VectorSubcoreMesh(core_axis_name="c", subcore_axis_name="s")

    @functools.partial(
        pl.kernel,
        mesh=mesh,
        compiler_params=pltpu.CompilerParams(use_tc_tiling_on_sc=False),
        out_type=[jax.ShapeDtypeStruct((D, L), jnp.float32),
                  jax.ShapeDtypeStruct((D,), jnp.int32)],
        scratch_types=[
            pltpu.VMEM((b_per_w,), jnp.int32),
            pltpu.VMEM((b_per_w, L), jnp.float32),
            pltpu.SemaphoreType.DMA,
        ],
    )
    def gather_kernel(table_hbm, ide_hbm, ido_hbm, out_hbm, idx_hbm,
                      idx_v, rows_v, sem):
        wid = lax.axis_index("s") * info.num_cores + lax.axis_index("c")
        base = wid * b_per_w
        for c in range(n_chunks):
            r = base + c * _SC_CHUNK
            pair = (c * _SC_CHUNK) // (2 * half)
            within = (c * _SC_CHUNK) % (2 * half)
            src = ide_hbm if within < half else ido_hbm
            off = within % half
            start = wid * (b_per_w // 2) + pair * half + off
            pltpu.sync_copy(src.at[pl.ds(start, _SC_CHUNK)],
                            idx_v.at[pl.ds(c * _SC_CHUNK, _SC_CHUNK)])
        copies = [
            pltpu.async_copy(
                table_hbm.at[idx_v.at[pl.ds(c * _SC_CHUNK, _SC_CHUNK)]],
                rows_v.at[pl.ds(c * _SC_CHUNK, _SC_CHUNK)],
                sem,
            )
            for c in range(n_chunks)
        ]
        for cp in copies:
            cp.wait()
        pltpu.sync_copy(rows_v, out_hbm.at[pl.ds(base, b_per_w)])
        pltpu.sync_copy(idx_v, idx_hbm.at[pl.ds(base, b_per_w)])

    return gather_kernel(codebook_KL, ide, ido)


def kernel(x_DL, codebook_KL, training):
    xn_DL, ide, ido, half = _tc_argmax(x_DL, codebook_KL, block_d=512)
    z_DL, indices_D = _sc_gather(codebook_KL, ide, ido, half)
    return (z_DL, z_DL, xn_DL, indices_D)


# speedup vs baseline: 1.2783x; 1.2783x over previous
import functools

import jax
import jax.numpy as jnp
from jax import lax
from jax.experimental import pallas as pl
from jax.experimental.pallas import tpu as pltpu
from jax.experimental.pallas import tpu_sc as plsc


def _fold_sumsq(v, axis):
    s = v * v
    n = s.shape[axis]
    while n > 1:
        n //= 2
        if axis == 0:
            s = s[:n, :] + s[n:2 * n, :]
        else:
            s = s[:, :n] + s[:, n:2 * n]
    return s


def _argmax_reduce(s, io_ref, K):
    R = s.shape[0]
    TW = 128
    T = K // TW
    RC = 64
    outs = []
    for r0 in range(0, R, RC):
        m = s[r0:r0 + RC, 0:TW]
        tt = jnp.zeros((RC, TW), jnp.float32)
        for t in range(1, T):
            st = s[r0:r0 + RC, t * TW:(t + 1) * TW]
            upd = st > m
            m = jnp.where(upd, st, m)
            tt = jnp.where(upd, float(t), tt)
        g = tt * float(TW) + io_ref[:, :TW]
        mm = jnp.max(m, axis=1, keepdims=True)
        idxf = jnp.min(jnp.where(m == mm, g, float(K)), axis=1)
        outs.append(jnp.minimum(idxf, K - 1).astype(jnp.int32))
    return jnp.concatenate(outs, axis=0)


def _argmax_body(x_ref, cbT_ref, xn_ref, idx_ref, bs_ref, io_ref):
    K = cbT_ref.shape[1]

    @pl.when(pl.program_id(0) == 0)
    def _prep():
        c = cbT_ref[...]
        cn = c / (jnp.sqrt(_fold_sumsq(c, 0)) + 1e-8)
        bs_ref[...] = cn.astype(jnp.bfloat16)
        io_ref[...] = lax.broadcasted_iota(jnp.int32, (1, K),
                                           1).astype(jnp.float32)

    x = x_ref[...]
    xn = x / (jnp.sqrt(_fold_sumsq(x, 1)) + 1e-8)
    xn_ref[...] = xn

    s = lax.dot_general(xn.astype(jnp.bfloat16), bs_ref[...],
                        (((1,), (0,)), ((), ())),
                        preferred_element_type=jnp.float32)
    idx_ref[...] = _argmax_reduce(s, io_ref, K)


def _tc_argmax(x_DL, codebook_KL, block_d):
    D, L = x_DL.shape
    K = codebook_KL.shape[0]
    return pl.pallas_call(
        _argmax_body,
        grid=(D // block_d,),
        in_specs=[
            pl.BlockSpec((block_d, L), lambda i: (i, 0)),
            pl.BlockSpec((L, K), lambda i: (0, 0)),
        ],
        out_specs=[
            pl.BlockSpec((block_d, L), lambda i: (i, 0)),
            pl.BlockSpec((block_d,), lambda i: (i,)),
        ],
        out_shape=[
            jax.ShapeDtypeStruct((D, L), jnp.float32),
            jax.ShapeDtypeStruct((D,), jnp.int32),
        ],
        scratch_shapes=[pltpu.VMEM((L, K), jnp.bfloat16),
                        pltpu.VMEM((1, K), jnp.float32)],
    )(x_DL, codebook_KL.T)


_SC_CHUNK = 128


def _sc_gather(codebook_KL, indices_D):
    D = indices_D.shape[0]
    L = codebook_KL.shape[1]
    info = plsc.get_sparse_core_info()
    nw = info.num_cores * info.num_subcores
    b_per_w = D // nw
    n_chunks = b_per_w // _SC_CHUNK
    mesh = plsc.VectorSubcoreMesh(core_axis_name="c", subcore_axis_name="s")

    @functools.partial(
        pl.kernel,
        mesh=mesh,
        compiler_params=pltpu.CompilerParams(use_tc_tiling_on_sc=False),
        out_type=jax.ShapeDtypeStruct((D, L), jnp.float32),
        scratch_types=[
            pltpu.VMEM((b_per_w,), jnp.int32),
            pltpu.VMEM((b_per_w, L), jnp.float32),
            pltpu.SemaphoreType.DMA,
        ],
    )
    def gather_kernel(table_hbm, idx_hbm, out_hbm, idx_v, rows_v, sem):
        wid = lax.axis_index("s") * info.num_cores + lax.axis_index("c")
        base = wid * b_per_w
        pltpu.sync_copy(idx_hbm.at[pl.ds(base, b_per_w)], idx_v)
        copies = [
            pltpu.async_copy(
                table_hbm.at[idx_v.at[pl.ds(c * _SC_CHUNK, _SC_CHUNK)]],
                rows_v.at[pl.ds(c * _SC_CHUNK, _SC_CHUNK)],
                sem,
            )
            for c in range(n_chunks)
        ]
        for cp in copies:
            cp.wait()
        pltpu.sync_copy(rows_v, out_hbm.at[pl.ds(base, b_per_w)])

    return gather_kernel(codebook_KL, indices_D)


def kernel(x_DL, codebook_KL, training):
    xn_DL, indices_D = _tc_argmax(x_DL, codebook_KL, block_d=512)
    z_DL = _sc_gather(codebook_KL, indices_D)
    return (z_DL, z_DL, xn_DL, indices_D)
